# Initial kernel scaffold; baseline (speedup 1.0000x reference)
import jax
import jax.numpy as jnp
from jax.experimental import pallas as pl


def kernel(x, switch_w, switch_b, w1, b1, w2, b2):
    raise NotImplementedError("write your pallas kernel here")


# speedup vs baseline: 10.0719x; 10.0719x over previous
import functools

import jax
import jax.numpy as jnp
from jax import lax
from jax.experimental import pallas as pl
from jax.experimental.pallas import tpu as pltpu
from jax.experimental.pallas import tpu_sc as plsc

B, S, D, F, E = 2, 8192, 768, 2048, 64
N = B * S
TM = 128
FBLK = 512
NF = F // FBLK
T = N // TM
P = T + E
NP = P * TM
RB = 512
NB = N // RB
SQRT1_2 = 0.7071067811865476


def _router_body(x_ref, sw_ref, sb_ref, routes_ref, pmax_ref):
    xb = x_ref[...]
    logits = lax.dot_general(xb, sw_ref[...], (((1,), (1,)), ((), ())),
                             preferred_element_type=jnp.float32)
    logits = logits + sb_ref[0][None, :]
    m = jnp.max(logits, axis=-1, keepdims=True)
    denom = jnp.sum(jnp.exp(logits - m), axis=-1)
    pmax_ref[0, 0, :] = 1.0 / denom
    ids = lax.broadcasted_iota(jnp.int32, logits.shape, 1)
    routes_ref[0, 0, :] = jnp.min(jnp.where(logits == m, ids, E), axis=-1)


def _ffn_body(eb_ref, x_ref, w1_ref, b1_ref, w2_ref, b2_ref, pr_ref, o_ref):
    f = pl.program_id(1)
    xb = x_ref[...]
    h = lax.dot_general(xb, w1_ref[0], (((1,), (1,)), ((), ())),
                        preferred_element_type=jnp.float32)
    h = h + b1_ref[0, 0][None, :]
    g = 0.5 * h * (1.0 + lax.erf(h * SQRT1_2))
    contrib = lax.dot_general(g, w2_ref[0], (((1,), (1,)), ((), ())),
                              preferred_element_type=jnp.float32)

    @pl.when(f == 0)
    def _():
        o_ref[...] = contrib + b2_ref[0, 0][None, :]

    @pl.when(f > 0)
    def _():
        o_ref[...] += contrib

    @pl.when(f == NF - 1)
    def _():
        o_ref[...] *= pr_ref[0, 0][:, None]


def _route_tokens(flat, switch_w, switch_b):
    routes2, pmax2 = pl.pallas_call(
        _router_body,
        grid=(NB,),
        in_specs=[
            pl.BlockSpec((RB, D), lambda i: (i, 0)),
            pl.BlockSpec((E, D), lambda i: (0, 0)),
            pl.BlockSpec((1, E), lambda i: (0, 0)),
        ],
        out_specs=[
            pl.BlockSpec((1, 1, RB), lambda i: (i, 0, 0)),
            pl.BlockSpec((1, 1, RB), lambda i: (i, 0, 0)),
        ],
        out_shape=[
            jax.ShapeDtypeStruct((NB, 1, RB), jnp.int32),
            jax.ShapeDtypeStruct((NB, 1, RB), jnp.float32),
        ],
    )(flat, switch_w, switch_b.reshape(1, E))
    return routes2.reshape(N), pmax2.reshape(N)


def _grouped_ffn(eblk, xg, w1, b1, w2, b2, pg2):
    grid_spec = pltpu.PrefetchScalarGridSpec(
        num_scalar_prefetch=1,
        grid=(P, NF),
        in_specs=[
            pl.BlockSpec((TM, D), lambda p, f, eb: (p, 0)),
            pl.BlockSpec((1, FBLK, D), lambda p, f, eb: (eb[p], f, 0)),
            pl.BlockSpec((1, 1, FBLK), lambda p, f, eb: (eb[p] * NF + f, 0, 0)),
            pl.BlockSpec((1, D, FBLK), lambda p, f, eb: (eb[p], 0, f)),
            pl.BlockSpec((1, 1, D), lambda p, f, eb: (eb[p], 0, 0)),
            pl.BlockSpec((1, 1, TM), lambda p, f, eb: (p, 0, 0)),
        ],
        out_specs=pl.BlockSpec((TM, D), lambda p, f, eb: (p, 0)),
    )
    return pl.pallas_call(
        _ffn_body,
        grid_spec=grid_spec,
        out_shape=jax.ShapeDtypeStruct((NP, D), jnp.float32),
        compiler_params=pltpu.CompilerParams(
            dimension_semantics=("arbitrary", "arbitrary")),
    )(eblk, xg, w1, b1.reshape(E * NF, 1, FBLK), w2, b2.reshape(E, 1, D), pg2)


def _sc_gather_with_p(flat, pmax, sidx):
    info = plsc.get_sparse_core_info()
    nc, ns = info.num_cores, info.num_subcores
    nw = nc * ns
    per_w = NP // nw
    ch = 128
    mesh = plsc.VectorSubcoreMesh(core_axis_name="c", subcore_axis_name="s")

    @functools.partial(
        pl.kernel, mesh=mesh,
        out_type=[
            jax.ShapeDtypeStruct((NP, D), jnp.float32),
            jax.ShapeDtypeStruct((NP,), jnp.float32),
        ],
        scratch_types=[
            pltpu.VMEM((ch,), jnp.int32),
            pltpu.VMEM((ch, D), jnp.float32),
            pltpu.VMEM((ch,), jnp.float32),
            pltpu.SemaphoreType.DMA,
        ],
    )
    def gk(flat_hbm, pmax_hbm, sidx_hbm, xg_hbm, pg_hbm,
           idx_v, rows_v, pbuf, sem):
        wid = lax.axis_index("s") * nc + lax.axis_index("c")
        base0 = wid * per_w
        for c in range(per_w // ch):
            base = base0 + c * ch
            pltpu.sync_copy(sidx_hbm.at[pl.ds(base, ch)], idx_v)
            pltpu.async_copy(flat_hbm.at[idx_v], rows_v, sem).wait()
            pltpu.sync_copy(rows_v, xg_hbm.at[pl.ds(base, ch)])
            pltpu.async_copy(pmax_hbm.at[idx_v], pbuf, sem).wait()
            pltpu.sync_copy(pbuf, pg_hbm.at[pl.ds(base, ch)])

    return gk(flat, pmax, sidx)


def _sc_gather_rows(src, idx, n_out):
    info = plsc.get_sparse_core_info()
    nc, ns = info.num_cores, info.num_subcores
    nw = nc * ns
    per_w = n_out // nw
    ch = 128
    mesh = plsc.VectorSubcoreMesh(core_axis_name="c", subcore_axis_name="s")

    @functools.partial(
        pl.kernel, mesh=mesh,
        out_type=jax.ShapeDtypeStruct((n_out, D), jnp.float32),
        scratch_types=[
            pltpu.VMEM((ch,), jnp.int32),
            pltpu.VMEM((ch, D), jnp.float32),
            pltpu.SemaphoreType.DMA,
        ],
    )
    def gk(src_hbm, idx_hbm, out_hbm, idx_v, rows_v, sem):
        wid = lax.axis_index("s") * nc + lax.axis_index("c")
        base0 = wid * per_w
        for c in range(per_w // ch):
            base = base0 + c * ch
            pltpu.sync_copy(idx_hbm.at[pl.ds(base, ch)], idx_v)
            pltpu.async_copy(src_hbm.at[idx_v], rows_v, sem).wait()
            pltpu.sync_copy(rows_v, out_hbm.at[pl.ds(base, ch)])

    return gk(src, idx)


def kernel(x, switch_w, switch_b, w1, b1, w2, b2):
    flat = x.reshape(N, D)

    routes, pmax = _route_tokens(flat, switch_w, switch_b)

    perm = jnp.argsort(routes)
    counts = jnp.bincount(routes, length=E)
    offs = jnp.concatenate([jnp.zeros(1, jnp.int32),
                            jnp.cumsum(counts).astype(jnp.int32)])
    ptiles = (counts + TM - 1) // TM
    bcum = jnp.cumsum(ptiles)
    bstart = (bcum - ptiles).astype(jnp.int32)
    eblk = jnp.minimum(
        jnp.searchsorted(bcum, jnp.arange(P), side="right"), E - 1
    ).astype(jnp.int32)
    slots = jnp.arange(NP, dtype=jnp.int32)
    e_i = eblk[slots // TM]
    r = slots - TM * bstart[e_i]
    tok_q = offs[e_i] + r
    valid = r < counts[e_i]
    sidx = jnp.where(valid, perm[jnp.minimum(tok_q, N - 1)], 0).astype(jnp.int32)
    q = jnp.arange(N, dtype=jnp.int32)
    e_q = jnp.searchsorted(offs[1:], q, side="right").astype(jnp.int32)
    slot_q = TM * bstart[e_q] + (q - offs[e_q])
    inv = jnp.zeros(N, jnp.int32).at[perm].set(slot_q)

    xg, pg = _sc_gather_with_p(flat, pmax, sidx)

    ys = _grouped_ffn(eblk, xg, w1, b1, w2, b2, pg.reshape(P, 1, TM))

    final = _sc_gather_rows(ys, inv, N)
    return final.reshape(B, S, D)
